# Initial kernel scaffold; baseline (speedup 1.0000x reference)
import jax
import jax.numpy as jnp
from jax.experimental import pallas as pl


def kernel(h_id_tensor, r_id_tensor, t_id_tensor, q_emb, entity_embs, num_non_text_entities, relation_embs, topic_entity_one_hot, non_text_emb, fc1_w_mu, fc1_w_rho, fc1_b_mu, fc1_b_rho, fc2_w_mu, fc2_w_rho, fc2_b_mu, fc2_b_rho):
    raise NotImplementedError("write your pallas kernel here")


# speedup vs baseline: 3.3754x; 3.3754x over previous
import functools

import jax
import jax.numpy as jnp
from jax import lax
from jax.experimental import pallas as pl
from jax.experimental.pallas import tpu as pltpu
from jax.experimental.pallas import tpu_sc as plsc

F32 = jnp.float32
I32 = jnp.int32

_N = 10000
_NP = 10240
_E = 160000
_NREL = 512
_D = 640
_NW = 32
_RANGE = _NP // _NW
_EPW = _E // _NW
_NBLK = 314
_IPW = _NBLK * 16
_ECH = 16000


def _mesh():
    return plsc.VectorSubcoreMesh(core_axis_name="c", subcore_axis_name="s")


_SC_PARAMS = pltpu.CompilerParams(needs_layout_passes=False)


def _wid():
    return lax.axis_index("c") * 16 + lax.axis_index("s")


def _zero_accs(accs):
    z = jnp.zeros((16,), F32)

    def body(k, _):
        j = k // (_RANGE // 16)
        c = k % (_RANGE // 16)
        for a in accs:
            a[j, pl.ds(c * 16, 16)] = z
        return 0

    lax.fori_loop(0, 16 * (_RANGE // 16), body, 0)


def _acc_total(a, sl):
    s = a[0, sl]
    for j in range(1, 16):
        s = s + a[j, sl]
    return s


def _edge_scan(h_hbm, t_hbm, hbuf, tbuf, lo, per16):
    def chunk_body(c, _):
        pltpu.sync_copy(h_hbm.at[pl.ds(c * _ECH, _ECH)], hbuf)
        pltpu.sync_copy(t_hbm.at[pl.ds(c * _ECH, _ECH)], tbuf)

        def eb(i, _2):
            h16 = hbuf[pl.ds(i * 16, 16)]
            t16 = tbuf[pl.ds(i * 16, 16)]
            lt = t16 - lo
            lh = h16 - lo
            mT = (lt >= 0) & (lt < _RANGE)
            mH = (lh >= 0) & (lh < _RANGE)
            ltc = jnp.clip(lt, 0, _RANGE - 1)
            lhc = jnp.clip(lh, 0, _RANGE - 1)
            per16(h16, t16, ltc, mT, lhc, mH)
            return 0

        lax.fori_loop(0, _ECH // 16, eb, 0, unroll=4)
        return 0

    lax.fori_loop(0, _E // _ECH, chunk_body, 0)


def _round1_kernel(h_ids, t_ids, x0, x1):
    def body(h_hbm, t_hbm, x0_hbm, x1_hbm,
             of0, of1, og0, og1, oct_, och_,
             x0v, x1v, hbuf, tbuf,
             aF0, aF1, aR0, aR1, aCT, aCH,
             sF0, sF1, sR0, sR1, sCT, sCH):
        w = _wid()
        lo = w * _RANGE
        pltpu.sync_copy(x0_hbm, x0v)
        pltpu.sync_copy(x1_hbm, x1v)
        _zero_accs([aF0, aF1, aR0, aR1, aCT, aCH])
        lane = lax.iota(I32, 16)
        ones = jnp.ones((16,), F32)

        def per16(h16, t16, ltc, mT, lhc, mH):
            xh0 = plsc.load_gather(x0v, [h16])
            xh1 = plsc.load_gather(x1v, [h16])
            xt0 = plsc.load_gather(x0v, [t16])
            xt1 = plsc.load_gather(x1v, [t16])
            plsc.addupdate_scatter(aF0, [lane, ltc], xh0, mask=mT)
            plsc.addupdate_scatter(aF1, [lane, ltc], xh1, mask=mT)
            plsc.addupdate_scatter(aCT, [lane, ltc], ones, mask=mT)
            plsc.addupdate_scatter(aR0, [lane, lhc], xt0, mask=mH)
            plsc.addupdate_scatter(aR1, [lane, lhc], xt1, mask=mH)
            plsc.addupdate_scatter(aCH, [lane, lhc], ones, mask=mH)

        _edge_scan(h_hbm, t_hbm, hbuf, tbuf, lo, per16)

        def red(c, _):
            sl = pl.ds(c * 16, 16)
            cT = _acc_total(aCT, sl)
            cH = _acc_total(aCH, sl)
            dT = jnp.maximum(cT, 1.0)
            dH = jnp.maximum(cH, 1.0)
            sF0[sl] = _acc_total(aF0, sl) / dT
            sF1[sl] = _acc_total(aF1, sl) / dT
            sR0[sl] = _acc_total(aR0, sl) / dH
            sR1[sl] = _acc_total(aR1, sl) / dH
            sCT[sl] = cT
            sCH[sl] = cH
            return 0

        lax.fori_loop(0, _RANGE // 16, red, 0)
        for stg, o in ((sF0, of0), (sF1, of1), (sR0, og0), (sR1, og1),
                       (sCT, oct_), (sCH, och_)):
            pltpu.sync_copy(stg, o.at[pl.ds(lo, _RANGE)])

    f = pl.kernel(
        body,
        out_type=[jax.ShapeDtypeStruct((_NP,), F32)] * 6,
        mesh=_mesh(),
        compiler_params=_SC_PARAMS,
        scratch_types=(
            [pltpu.VMEM((_NP,), F32)] * 2
            + [pltpu.VMEM((_ECH,), I32)] * 2
            + [pltpu.VMEM((16, _RANGE), F32)] * 6
            + [pltpu.VMEM((_RANGE,), F32)] * 6
        ),
    )
    return f(h_ids, t_ids, x0, x1)


def _round2_kernel(h_ids, t_ids, f0, f1, g0, g1, ct, ch):
    def body(h_hbm, t_hbm, f0_hbm, f1_hbm, g0_hbm, g1_hbm, ct_hbm, ch_hbm,
             of0, of1, og0, og1,
             f0v, f1v, g0v, g1v, hbuf, tbuf,
             aF0, aF1, aR0, aR1,
             ctv, chv, sF0, sF1, sR0, sR1):
        w = _wid()
        lo = w * _RANGE
        pltpu.sync_copy(f0_hbm, f0v)
        pltpu.sync_copy(f1_hbm, f1v)
        pltpu.sync_copy(g0_hbm, g0v)
        pltpu.sync_copy(g1_hbm, g1v)
        pltpu.sync_copy(ct_hbm.at[pl.ds(lo, _RANGE)], ctv)
        pltpu.sync_copy(ch_hbm.at[pl.ds(lo, _RANGE)], chv)
        _zero_accs([aF0, aF1, aR0, aR1])
        lane = lax.iota(I32, 16)

        def per16(h16, t16, ltc, mT, lhc, mH):
            xh0 = plsc.load_gather(f0v, [h16])
            xh1 = plsc.load_gather(f1v, [h16])
            xt0 = plsc.load_gather(g0v, [t16])
            xt1 = plsc.load_gather(g1v, [t16])
            plsc.addupdate_scatter(aF0, [lane, ltc], xh0, mask=mT)
            plsc.addupdate_scatter(aF1, [lane, ltc], xh1, mask=mT)
            plsc.addupdate_scatter(aR0, [lane, lhc], xt0, mask=mH)
            plsc.addupdate_scatter(aR1, [lane, lhc], xt1, mask=mH)

        _edge_scan(h_hbm, t_hbm, hbuf, tbuf, lo, per16)

        def red(c, _):
            sl = pl.ds(c * 16, 16)
            dT = jnp.maximum(ctv[sl], 1.0)
            dH = jnp.maximum(chv[sl], 1.0)
            sF0[sl] = _acc_total(aF0, sl) / dT
            sF1[sl] = _acc_total(aF1, sl) / dT
            sR0[sl] = _acc_total(aR0, sl) / dH
            sR1[sl] = _acc_total(aR1, sl) / dH
            return 0

        lax.fori_loop(0, _RANGE // 16, red, 0)
        for stg, o in ((sF0, of0), (sF1, of1), (sR0, og0), (sR1, og1)):
            pltpu.sync_copy(stg, o.at[pl.ds(lo, _RANGE)])

    f = pl.kernel(
        body,
        out_type=[jax.ShapeDtypeStruct((_NP,), F32)] * 4,
        mesh=_mesh(),
        compiler_params=_SC_PARAMS,
        scratch_types=(
            [pltpu.VMEM((_NP,), F32)] * 4
            + [pltpu.VMEM((_ECH,), I32)] * 2
            + [pltpu.VMEM((16, _RANGE), F32)] * 4
            + [pltpu.VMEM((_RANGE,), F32)] * 6
        ),
    )
    return f(h_ids, t_ids, f0, f1, g0, g1, ct, ch)


def _edge_mlp_kernel(hp, rp, tp, A, B, C2, vcat):
    def body(hp_hbm, rp_hbm, tp_hbm, A_hbm, B_hbm, C_hbm, v_hbm,
             out_hbm,
             hv, rv, tv, a0, a1, b0, b1, c0b, c1b, vv, ob, s0, s1):
        w = _wid()
        pltpu.sync_copy(hp_hbm.at[w], hv)
        pltpu.sync_copy(rp_hbm.at[w], rv)
        pltpu.sync_copy(tp_hbm.at[w], tv)
        pltpu.sync_copy(v_hbm, vv)
        ab = (a0, a1)
        bb_ = (b0, b1)
        cb = (c0b, c1b)
        sems = (s0, s1)
        lane = lax.iota(I32, 16)

        def issue(blk, p):
            sl = pl.ds(blk * 16, 16)
            pltpu.async_copy(A_hbm.at[hv.at[sl]], ab[p], sems[p])
            pltpu.async_copy(B_hbm.at[tv.at[sl]], bb_[p], sems[p])
            pltpu.async_copy(C_hbm.at[rv.at[sl]], cb[p], sems[p])

        def wait(blk, p):
            sl = pl.ds(blk * 16, 16)
            pltpu.make_async_copy(A_hbm.at[hv.at[sl]], ab[p], sems[p]).wait()
            pltpu.make_async_copy(B_hbm.at[tv.at[sl]], bb_[p], sems[p]).wait()
            pltpu.make_async_copy(C_hbm.at[rv.at[sl]], cb[p], sems[p]).wait()

        def compute(blk, p):
            a, b_, c_ = ab[p], bb_[p], cb[p]

            def ch_body(ch, accs):
                sl = pl.ds(ch * 16, 16)
                v16 = vv[sl]
                out = []
                for e in range(16):
                    x = a[e, sl] + b_[e, sl] + c_[e, sl]
                    out.append(accs[e] + jnp.maximum(x, 0.0) * v16)
                return tuple(out)

            accs = lax.fori_loop(
                0, _D // 16, ch_body,
                tuple(jnp.zeros((16,), F32) for _ in range(16)), unroll=4)
            o16 = jnp.zeros((16,), F32)
            for e in range(16):
                o16 = jnp.where(lane == e, jnp.sum(accs[e]), o16)
            ob[pl.ds(blk * 16, 16)] = o16

        issue(0, 0)

        def bb_body(i, _):
            blk = i * 2
            wait(blk, 0)
            issue(blk + 1, 1)
            compute(blk, 0)
            wait(blk + 1, 1)
            issue(blk + 2, 0)
            compute(blk + 1, 1)
            return 0

        lax.fori_loop(0, (_NBLK - 2) // 2, bb_body, 0)
        wait(_NBLK - 2, 0)
        issue(_NBLK - 1, 1)
        compute(_NBLK - 2, 0)
        wait(_NBLK - 1, 1)
        compute(_NBLK - 1, 1)
        pltpu.sync_copy(ob, out_hbm.at[w])

    f = pl.kernel(
        body,
        out_type=jax.ShapeDtypeStruct((_NW, _IPW), F32),
        mesh=_mesh(),
        compiler_params=_SC_PARAMS,
        scratch_types=(
            [pltpu.VMEM((_IPW,), I32)] * 3
            + [pltpu.VMEM((16, _D), F32)] * 6
            + [pltpu.VMEM((_D,), F32), pltpu.VMEM((_IPW,), F32),
               pltpu.SemaphoreType.DMA, pltpu.SemaphoreType.DMA]
        ),
    )
    return f(hp, rp, tp, A, B, C2, vcat)


def _proj_ab(h_e_pad, WhT, WtT):
    def body(x_ref, wh_ref, wt_ref, a_ref, b_ref):
        x = x_ref[...]
        a_ref[...] = jnp.dot(x, wh_ref[...], preferred_element_type=F32)
        b_ref[...] = jnp.dot(x, wt_ref[...], preferred_element_type=F32)

    return pl.pallas_call(
        body,
        grid=(_NP // 512,),
        in_specs=[pl.BlockSpec((512, 144), lambda i: (i, 0)),
                  pl.BlockSpec((144, _D), lambda i: (0, 0)),
                  pl.BlockSpec((144, _D), lambda i: (0, 0))],
        out_specs=[pl.BlockSpec((512, _D), lambda i: (i, 0)),
                   pl.BlockSpec((512, _D), lambda i: (i, 0))],
        out_shape=[jax.ShapeDtypeStruct((_NP, _D), F32)] * 2,
    )(h_e_pad, WhT, WtT)


def _proj_c(rel, WrT, qpad, WqT, bpad):
    def body(r_ref, wr_ref, q_ref, wq_ref, b_ref, o_ref):
        qrow = jnp.dot(q_ref[...], wq_ref[...], preferred_element_type=F32)
        o_ref[...] = (jnp.dot(r_ref[...], wr_ref[...], preferred_element_type=F32)
                      + qrow[0:1, :] + b_ref[0:1, :])

    return pl.pallas_call(
        body,
        out_shape=jax.ShapeDtypeStruct((_NREL, _D), F32),
    )(rel, WrT, qpad, WqT, bpad)


def kernel(h_id_tensor, r_id_tensor, t_id_tensor, q_emb, entity_embs,
           num_non_text_entities, relation_embs, topic_entity_one_hot,
           non_text_emb, fc1_w_mu, fc1_w_rho, fc1_b_mu, fc1_b_rho,
           fc2_w_mu, fc2_w_rho, fc2_b_mu, fc2_b_rho):
    h_ids = h_id_tensor.astype(I32)
    r_ids = r_id_tensor.astype(I32)
    t_ids = t_id_tensor.astype(I32)

    w1_sigma = jax.nn.softplus(fc1_w_rho)
    b1_sigma = jax.nn.softplus(fc1_b_rho)
    w2_sigma = jax.nn.softplus(fc2_w_rho)
    b2_sigma = jax.nn.softplus(fc2_b_rho)
    mc_key = jax.random.key(42)
    W1s, b1s, w2s, b2s = [], [], [], []
    for s in range(5):
        k1, k2 = jax.random.split(jax.random.fold_in(mc_key, s))
        k11, k12 = jax.random.split(k1)
        k21, k22 = jax.random.split(k2)
        W1s.append(fc1_w_mu + jax.random.normal(k11, fc1_w_mu.shape, F32) * w1_sigma)
        b1s.append(fc1_b_mu + jax.random.normal(k12, fc1_b_mu.shape, F32) * b1_sigma)
        w2s.append(fc2_w_mu + jax.random.normal(k21, fc2_w_mu.shape, F32) * w2_sigma)
        b2s.append(fc2_b_mu + jax.random.normal(k22, fc2_b_mu.shape, F32) * b2_sigma)
    Wcat = jnp.concatenate(W1s, axis=0)
    bcat = jnp.concatenate(b1s)
    vcat = jnp.concatenate([w.reshape(-1) for w in w2s]) / 5.0
    c0 = jnp.mean(jnp.stack([b.reshape(()) for b in b2s]))

    WqT = Wcat[:, 0:128].T
    WhT = jnp.pad(Wcat[:, 128:266].T, ((0, 6), (0, 0)))
    WrT = Wcat[:, 266:394].T
    WtT = jnp.pad(Wcat[:, 394:532].T, ((0, 6), (0, 0)))

    topic = topic_entity_one_hot.astype(F32)
    x0 = jnp.pad(topic[:, 0], (0, _NP - _N))
    x1 = jnp.pad(topic[:, 1], (0, _NP - _N))
    d10, d11, r10, r11, cnt_t, cnt_h = _round1_kernel(h_ids, t_ids, x0, x1)
    d20, d21, r20, r21 = _round2_kernel(h_ids, t_ids, d10, d11, r10, r11,
                                        cnt_t, cnt_h)

    nnt = topic.shape[0] - entity_embs.shape[0]
    ntu = non_text_emb + (jnp.asarray(num_non_text_entities) - nnt).astype(F32)
    h_e0 = jnp.concatenate(
        [entity_embs, jnp.broadcast_to(ntu, (nnt, non_text_emb.shape[-1]))], axis=0)
    he = jnp.concatenate(
        [h_e0, topic,
         jnp.stack([d10[:_N], d11[:_N]], axis=1),
         jnp.stack([d20[:_N], d21[:_N]], axis=1),
         jnp.stack([r10[:_N], r11[:_N]], axis=1),
         jnp.stack([r20[:_N], r21[:_N]], axis=1)], axis=1)
    h_e_pad = jnp.pad(he, ((0, _NP - _N), (0, 144 - 138)))

    A, B = _proj_ab(h_e_pad, WhT, WtT)
    qpad = jnp.pad(q_emb.astype(F32), ((0, 7), (0, 0)))
    bpad = jnp.pad(bcat[None, :], ((0, 7), (0, 0)))
    C2 = _proj_c(relation_embs.astype(F32), WrT, qpad, WqT, bpad)

    def split_pad(ids):
        return jnp.pad(ids.reshape(_NW, _EPW), ((0, 0), (0, _IPW - _EPW)))

    out2 = _edge_mlp_kernel(split_pad(h_ids), split_pad(r_ids),
                            split_pad(t_ids), A, B, C2, vcat)
    out = out2[:, :_EPW].reshape(_E, 1) + c0
    return out.astype(F32)
